# Initial kernel scaffold; baseline (speedup 1.0000x reference)
import jax
import jax.numpy as jnp
from jax.experimental import pallas as pl


def kernel(input_ids, token_embeddings, position_embeddings):
    raise NotImplementedError("write your pallas kernel here")


# speedup vs baseline: 1.0080x; 1.0080x over previous
import functools

import jax
import jax.numpy as jnp
from jax import lax
from jax.experimental import pallas as pl
from jax.experimental.pallas import tpu as pltpu
from jax.experimental.pallas import tpu_sc as plsc

_VOCAB = 50257
_SEQLEN = 2048
_EMBED = 768
_BATCH = 4

_NUM_WORKERS = 32
_TOKENS = _BATCH * _SEQLEN
_BPW = _TOKENS // _NUM_WORKERS
_CHUNK = 64
_NCHUNKS = _BPW // _CHUNK
_LANES = 16


def _emb_body(ids_hbm, wte_hbm, wpe_hbm, out_hbm, idx_v, rows_v, pos_v, sem):
    wid = lax.axis_index("s") * 2 + lax.axis_index("c")
    base = wid * _BPW
    pos_base = lax.rem(base, _SEQLEN)
    for c in range(_NCHUNKS):
        off = c * _CHUNK
        pltpu.sync_copy(ids_hbm.at[pl.ds(base + off, _CHUNK)], idx_v)
        gat = pltpu.async_copy(wte_hbm.at[idx_v], rows_v, sem)
        pltpu.sync_copy(wpe_hbm.at[pl.ds(pos_base + off, _CHUNK)], pos_v)
        gat.wait()

        def add_row(r, carry):
            for k in range(_EMBED // _LANES):
                sl = pl.ds(k * _LANES, _LANES)
                rows_v[r, sl] = rows_v[r, sl] + pos_v[r, sl]
            return carry

        lax.fori_loop(0, _CHUNK, add_row, 0)
        pltpu.sync_copy(rows_v, out_hbm.at[pl.ds(base + off, _CHUNK)])


@jax.jit
def kernel(input_ids, token_embeddings, position_embeddings):
    ids_flat = input_ids.reshape(_TOKENS)
    mesh = plsc.VectorSubcoreMesh(core_axis_name="c", subcore_axis_name="s")
    out = pl.kernel(
        _emb_body,
        out_type=jax.ShapeDtypeStruct((_TOKENS, _EMBED), jnp.float32),
        mesh=mesh,
        scratch_types=[
            pltpu.VMEM((_CHUNK,), jnp.int32),
            pltpu.VMEM((_CHUNK, _EMBED), jnp.float32),
            pltpu.VMEM((_CHUNK, _EMBED), jnp.float32),
            pltpu.SemaphoreType.DMA,
        ],
    )(ids_flat, token_embeddings, position_embeddings)
    return out.reshape(_BATCH, _SEQLEN, _EMBED)
